# baseline (speedup 1.0000x reference)
import jax
import jax.numpy as jnp
from jax.experimental import pallas as pl
from jax.experimental.pallas import tpu as pltpu

_NCLS = 1000
_MAXP = 100
_FDIM = 128


def _body(cid_ref, feat_hbm, protos_in, counts_in, protos_out, counts_out,
          featv, normv, onesv, sem_f, sem_row, sem_cnt):
    cid = cid_ref[0]

    feat_in = pltpu.make_async_copy(feat_hbm.at[pl.ds(0, 104)], featv, sem_f)
    feat_in.start()
    onesv[...] = jnp.ones((8, _MAXP), jnp.int32)
    feat_in.wait()

    f = featv[...]
    norm = jnp.sqrt(jnp.sum(f * f, axis=1, keepdims=True))
    normv[...] = (f / jnp.maximum(norm, 1e-12))[:_MAXP]

    row_wr = pltpu.make_async_copy(normv, protos_out.at[cid], sem_row)
    cnt_wr = pltpu.make_async_copy(
        onesv.at[pl.ds(0, 1)], counts_out.at[pl.ds(cid, 1)], sem_cnt)
    row_wr.start()
    cnt_wr.start()
    row_wr.wait()
    cnt_wr.wait()


def kernel(features, prototypes, counts, class_id):
    cid = jnp.atleast_1d(jnp.asarray(class_id, jnp.int32))
    grid_spec = pltpu.PrefetchScalarGridSpec(
        num_scalar_prefetch=1,
        grid=(1,),
        in_specs=[
            pl.BlockSpec(memory_space=pltpu.MemorySpace.HBM),
            pl.BlockSpec(memory_space=pltpu.MemorySpace.HBM),
            pl.BlockSpec(memory_space=pltpu.MemorySpace.HBM),
        ],
        out_specs=[
            pl.BlockSpec(memory_space=pltpu.MemorySpace.HBM),
            pl.BlockSpec(memory_space=pltpu.MemorySpace.HBM),
        ],
        scratch_shapes=[
            pltpu.VMEM((104, _FDIM), jnp.float32),
            pltpu.VMEM((_MAXP, _FDIM), jnp.float32),
            pltpu.VMEM((8, _MAXP), jnp.int32),
            pltpu.SemaphoreType.DMA,
            pltpu.SemaphoreType.DMA,
            pltpu.SemaphoreType.DMA,
        ],
    )
    return pl.pallas_call(
        _body,
        grid_spec=grid_spec,
        out_shape=(
            jax.ShapeDtypeStruct((_NCLS, _MAXP, _FDIM), jnp.float32),
            jax.ShapeDtypeStruct((_NCLS, _MAXP), jnp.int32),
        ),
        input_output_aliases={2: 0, 3: 1},
        compiler_params=pltpu.CompilerParams(
            dimension_semantics=("arbitrary",),
        ),
    )(cid, features, prototypes, counts)
